# Initial kernel scaffold; baseline (speedup 1.0000x reference)
import jax
import jax.numpy as jnp
from jax.experimental import pallas as pl


def kernel(vfts, adjs, rels, embed_rels, embed_rels_target, weight, comp, attn_w1, attn_b1, attn_w2, attn_b2, update):
    raise NotImplementedError("write your pallas kernel here")


# speedup vs baseline: 9.6394x; 9.6394x over previous
import functools

import jax
import jax.numpy as jnp
from jax import lax
from jax.experimental import pallas as pl
from jax.experimental.pallas import tpu as pltpu
from jax.experimental.pallas import tpu_sc as plsc

_NCORE = 2
_NSUB = 16
_K = 80
_HALF = 5120
_NPAD = 10240


def _mm_body(alpha_ref, comp_ref, x_ref, w_ref, upd_ref, o1_ref, o2_ref):
    r = pl.program_id(1)
    x = x_ref[...]

    @pl.when(r < 16)
    def _():
        w = (comp_ref[r, 0] * w_ref[0] + comp_ref[r, 1] * w_ref[1]
             + comp_ref[r, 2] * w_ref[2] + comp_ref[r, 3] * w_ref[3])
        o1_ref[0] = alpha_ref[r, 0] * jnp.dot(
            x, w, preferred_element_type=jnp.float32)

    @pl.when(r == 16)
    def _():
        o2_ref[...] = jnp.dot(
            x, upd_ref[...], preferred_element_type=jnp.float32)


def _build_tables(alphas, comp, vfts, weight, update):
    n, d = vfts.shape
    nrel = comp.shape[0]
    nblk = 10
    b = n // nblk
    return pl.pallas_call(
        _mm_body,
        grid=(nblk, nrel + 1),
        in_specs=[
            pl.BlockSpec((nrel, 1), lambda i, r: (0, 0),
                         memory_space=pltpu.SMEM),
            pl.BlockSpec((nrel, 4), lambda i, r: (0, 0),
                         memory_space=pltpu.SMEM),
            pl.BlockSpec((b, d), lambda i, r: (i, 0)),
            pl.BlockSpec((4, d, d), lambda i, r: (0, 0, 0)),
            pl.BlockSpec((d, d), lambda i, r: (0, 0)),
        ],
        out_specs=[
            pl.BlockSpec((1, b, d), lambda i, r: (jnp.minimum(r, 15), i, 0)),
            pl.BlockSpec((b, d), lambda i, r: (i, 0)),
        ],
        out_shape=[
            jax.ShapeDtypeStruct((nrel, n, d), jnp.float32),
            jax.ShapeDtypeStruct((n, d), jnp.float32),
        ],
        compiler_params=pltpu.CompilerParams(
            dimension_semantics=("arbitrary", "arbitrary")),
    )(alphas, comp, vfts, weight, update)


_SEG = 50


def _make_sc(n, d, e):
    ept = e // _NSUB
    seg = _SEG * _K
    nseg = ept // seg
    rpt = _HALF // _NSUB
    nko = rpt // _K
    mesh = plsc.VectorSubcoreMesh(
        core_axis_name="c", subcore_axis_name="s",
        num_cores=_NCORE, num_subcores=_NSUB)

    @functools.partial(
        pl.kernel,
        out_type=(
            jax.ShapeDtypeStruct((_NCORE, _HALF, d), jnp.float32),
            jax.ShapeDtypeStruct((_NSUB, 1, _NPAD), jnp.float32),
        ),
        mesh=mesh,
        scratch_types=[
            pltpu.VMEM((seg,), jnp.int32),
            pltpu.VMEM((seg,), jnp.int32),
            pltpu.VMEM((seg,), jnp.int32),
            pltpu.VMEM((seg,), jnp.int32),
            pltpu.VMEM((_SEG, _K), jnp.int32),
            pltpu.VMEM((_NPAD,), jnp.float32),
            pltpu.VMEM((2, _K, d), jnp.float32),
            pltpu.VMEM_SHARED((_HALF + 8, d), jnp.float32),
            pltpu.SemaphoreType.DMA,
            pltpu.SemaphoreType.DMA,
        ],
        compiler_params=pltpu.CompilerParams(needs_layout_passes=False),
    )
    def sc_kernel(tbl_hbm, rels_hbm, src_hbm, dst_hbm, zrow_hbm, zcnt_hbm,
                  acc_out, cnt_out,
                  rels_v, src_v, gidx_v, dst1_v, dst2_v, cnt_v, rows_v,
                  acc_sh, sem0, sem1):
        c = lax.axis_index("c")
        s = lax.axis_index("s")
        base_e = s * ept

        pltpu.sync_copy(zcnt_hbm, cnt_v)

        pltpu.sync_copy(zrow_hbm, rows_v.at[1])
        for k in range(nko):
            pltpu.sync_copy(rows_v.at[1],
                            acc_sh.at[pl.ds(s * rpt + k * _K, _K)])

        @pl.when(s == 0)
        def _():
            pltpu.sync_copy(rows_v.at[1, pl.ds(0, 8)],
                            acc_sh.at[pl.ds(_HALF, 8)])

        plsc.subcore_barrier()

        ones16 = jnp.ones((16,), jnp.float32)
        nvec = jnp.full((16,), n, jnp.int32)
        lovec = jnp.zeros((16,), jnp.int32) + c * _HALF
        trash = jnp.full((16,), _HALF, jnp.int32)

        def seg_body(g, carry):
            off = base_e + g * seg
            pltpu.sync_copy(rels_hbm.at[pl.ds(off, seg)], rels_v)
            pltpu.sync_copy(src_hbm.at[pl.ds(off, seg)], src_v)
            pltpu.sync_copy(dst_hbm.at[pl.ds(off, seg)], dst1_v)

            def idx_body(i, cr):
                sl = pl.ds(i * 16, 16)
                gidx_v[sl] = rels_v[sl] * nvec + src_v[sl]
                dv = dst1_v[sl]

                @pl.when(c == 0)
                def _():
                    plsc.addupdate_scatter(cnt_v, [dv], ones16)

                dl = dv - lovec
                oob = (dl < 0) | (dl >= _HALF)
                dst2_v[i // (_K // 16),
                       pl.ds((i % (_K // 16)) * 16, 16)] = jnp.where(
                           oob, trash, dl)
                return cr

            lax.fori_loop(0, seg // 16, idx_body, 0)

            def pair_body(jj, cr):
                j0 = 2 * jj
                j1 = j0 + 1
                g0 = pltpu.async_copy(
                    tbl_hbm.at[gidx_v.at[pl.ds(j0 * _K, _K)]],
                    rows_v.at[0], sem0)
                g1 = pltpu.async_copy(
                    tbl_hbm.at[gidx_v.at[pl.ds(j1 * _K, _K)]],
                    rows_v.at[1], sem1)
                g0.wait()
                pltpu.sync_copy(rows_v.at[0], acc_sh.at[dst2_v.at[j0]],
                                add=True)
                g1.wait()
                pltpu.sync_copy(rows_v.at[1], acc_sh.at[dst2_v.at[j1]],
                                add=True)
                return cr

            lax.fori_loop(0, _SEG // 2, pair_body, 0)
            return carry

        lax.fori_loop(0, nseg, seg_body, 0)

        plsc.subcore_barrier()

        for k in range(nko):
            pltpu.sync_copy(acc_sh.at[pl.ds(s * rpt + k * _K, _K)],
                            rows_v.at[1])
            pltpu.sync_copy(rows_v.at[1],
                            acc_out.at[c, pl.ds(s * rpt + k * _K, _K)])

        @pl.when(c == 0)
        def _():
            pltpu.sync_copy(cnt_v, cnt_out.at[s, 0])

    return sc_kernel


def _combine_body(b, acc_ref, cnt_ref, dst_ref, o_ref):
    i = pl.program_id(0)
    deg = jnp.sum(cnt_ref[:, pl.ds(i * b, b)], axis=0)
    o_ref[...] = jnp.maximum(acc_ref[...] + deg[:, None] * dst_ref[...], 0.0)


def _combine(acc, cnt, dsts):
    n, d = dsts.shape
    b = 512
    nblk = (n + b - 1) // b
    return pl.pallas_call(
        functools.partial(_combine_body, b),
        grid=(nblk,),
        in_specs=[
            pl.BlockSpec((b, d), lambda i: (i, 0)),
            pl.BlockSpec((_NSUB, _NPAD), lambda i: (0, 0)),
            pl.BlockSpec((b, d), lambda i: (i, 0)),
        ],
        out_specs=pl.BlockSpec((b, d), lambda i: (i, 0)),
        out_shape=jax.ShapeDtypeStruct((n, d), jnp.float32),
    )(acc, cnt, dsts)


def kernel(vfts, adjs, rels, embed_rels, embed_rels_target, weight, comp,
           attn_w1, attn_b1, attn_w2, attn_b2, update):
    n, d = vfts.shape
    e = rels.shape[0]
    nrel = comp.shape[0]
    assert e % (_NSUB * _SEG * _K) == 0 and _NCORE * _HALF >= n

    s16 = adjs[0, :nrel]
    t16 = adjs[1, :nrel]
    erps16 = jnp.concatenate(
        [vfts[s16], vfts[t16], embed_rels[:nrel],
         embed_rels_target[:nrel]], axis=1)
    h16 = jax.nn.relu(erps16 @ attn_w1.T + attn_b1)
    alphas = jax.nn.sigmoid(h16 @ attn_w2.T + attn_b2)

    tbl3, dsts = _build_tables(alphas, comp, vfts, weight, update)
    tbl = tbl3.reshape(nrel * n, d)

    zrow = jnp.zeros((_K, d), jnp.float32)
    zcnt = jnp.zeros((_NPAD,), jnp.float32)

    acc, cnt = _make_sc(n, d, e)(tbl, rels, adjs[0], adjs[1], zrow, zcnt)
    return _combine(acc.reshape(_NCORE * _HALF, d),
                    cnt.reshape(_NSUB, _NPAD), dsts)
